# baseline (speedup 1.0000x reference)
import functools

import jax
import jax.numpy as jnp
from jax.experimental import pallas as pl
from jax.experimental.pallas import tpu as pltpu


_NSTREAM = 4


def _ohem_body(x0_ref, x1_ref, x2_ref, x3_ref,
               t0_ref, t1_ref, t2_ref, t3_ref, out_ref,
               m_ref, s_ref, tx_ref, ts_ref, *,
               cc: int, h: int, w: int):
    j = pl.program_id(1)
    nj = pl.num_programs(1)

    @pl.when(j == 0)
    def _init():
        m_ref[...] = jnp.full((8, w), -jnp.inf, jnp.float32)
        s_ref[...] = jnp.zeros((8, w), jnp.float32)
        tx_ref[...] = jnp.zeros((8, w), jnp.float32)
        ts_ref[...] = jnp.zeros((8, w), jnp.float32)

    for xr, tr in ((x0_ref, t0_ref), (x1_ref, t1_ref),
                   (x2_ref, t2_ref), (x3_ref, t3_ref)):
        x = xr[0, 0].reshape(cc * h // 8, 8, w)
        t = tr[0, 0].reshape(cc * h // 8, 8, w)

        m_old = m_ref[...]
        m_new = jnp.maximum(m_old, jnp.max(x, axis=0))
        s_ref[...] = (s_ref[...] * jnp.exp(m_old - m_new)
                      + jnp.sum(jnp.exp(x - m_new[None]), axis=0))
        m_ref[...] = m_new
        tx_ref[...] += jnp.sum(t * x, axis=0)
        ts_ref[...] += jnp.sum(t, axis=0)

    @pl.when(j == nj - 1)
    def _finish():
        m_vec = m_ref[...]
        m_glob = jnp.max(m_vec)
        s_tot = jnp.sum(s_ref[...] * jnp.exp(m_vec - m_glob))
        loss = (jnp.sum(ts_ref[...]) * (m_glob + jnp.log(s_tot))
                - jnp.sum(tx_ref[...]))
        out_ref[0, 0, 0] = loss


def kernel(inputs, targets):
    batch, c, h, w = inputs.shape
    ns = _NSTREAM
    cs = c // ns
    cc = cs
    for cand in (6, 4, 3, 2, 1):
        if cs % cand == 0:
            cc = cand
            break
    nchunk = cs // cc

    x = inputs.reshape(batch, ns, cs, h, w)
    t = targets.reshape(batch, ns, cs, h, w)

    def spec(s):
        return pl.BlockSpec((1, 1, cc, h, w),
                            lambda b, j, s=s: (b, s, j, 0, 0))

    out = pl.pallas_call(
        functools.partial(_ohem_body, cc=cc, h=h, w=w),
        grid=(batch, nchunk),
        in_specs=[spec(0), spec(1), spec(2), spec(3),
                  spec(0), spec(1), spec(2), spec(3)],
        out_specs=pl.BlockSpec((1, 1, 1), lambda b, j: (b, 0, 0),
                               memory_space=pltpu.SMEM),
        out_shape=jax.ShapeDtypeStruct((batch, 1, 1), jnp.float32),
        scratch_shapes=[
            pltpu.VMEM((8, w), jnp.float32),
            pltpu.VMEM((8, w), jnp.float32),
            pltpu.VMEM((8, w), jnp.float32),
            pltpu.VMEM((8, w), jnp.float32),
        ],
        compiler_params=pltpu.CompilerParams(
            dimension_semantics=("parallel", "arbitrary")),
    )(x, x, x, x, t, t, t, t)
    return jnp.mean(out)


# speedup vs baseline: 1.0493x; 1.0493x over previous
import functools

import jax
import jax.numpy as jnp
from jax.experimental import pallas as pl
from jax.experimental.pallas import tpu as pltpu


_NSTREAM = 4


def _ohem_body(x0_ref, x1_ref, x2_ref, x3_ref,
               t0_ref, t1_ref, t2_ref, t3_ref, out_ref,
               m_ref, s_ref, tx_ref, ts_ref, *,
               cc: int, h: int, w: int):
    j = pl.program_id(1)
    nj = pl.num_programs(1)

    @pl.when(j == 0)
    def _init():
        m_ref[...] = jnp.full((8, w), -jnp.inf, jnp.float32)
        s_ref[...] = jnp.zeros((8, w), jnp.float32)
        tx_ref[...] = jnp.zeros((8, w), jnp.float32)
        ts_ref[...] = jnp.zeros((8, w), jnp.float32)

    for xr, tr in ((x0_ref, t0_ref), (x1_ref, t1_ref),
                   (x2_ref, t2_ref), (x3_ref, t3_ref)):
        x = xr[0, 0].reshape(cc * h // 8, 8, w)
        t = tr[0, 0].reshape(cc * h // 8, 8, w)

        m_old = m_ref[...]
        m_new = jnp.maximum(m_old, jnp.max(x, axis=0))
        s_ref[...] = (s_ref[...] * jnp.exp(m_old - m_new)
                      + jnp.sum(jnp.exp(x - m_new[None]), axis=0))
        m_ref[...] = m_new
        tx_ref[...] += jnp.sum(t * x, axis=0)
        ts_ref[...] += jnp.sum(t, axis=0)

    @pl.when(j == nj - 1)
    def _finish():
        m_vec = m_ref[...]
        m_glob = jnp.max(m_vec)
        s_tot = jnp.sum(s_ref[...] * jnp.exp(m_vec - m_glob))
        loss = (jnp.sum(ts_ref[...]) * (m_glob + jnp.log(s_tot))
                - jnp.sum(tx_ref[...]))
        out_ref[0, 0, 0] = loss


def kernel(inputs, targets):
    batch, c, h, w = inputs.shape
    ns = _NSTREAM
    cs = c // ns
    cc = cs
    for cand in (12, 8, 6, 4, 3, 2, 1):
        if cs % cand == 0:
            cc = cand
            break
    nchunk = cs // cc

    x = inputs.reshape(batch, ns, cs, h, w)
    t = targets.reshape(batch, ns, cs, h, w)

    def spec(s):
        return pl.BlockSpec((1, 1, cc, h, w),
                            lambda b, j, s=s: (b, s, j, 0, 0))

    out = pl.pallas_call(
        functools.partial(_ohem_body, cc=cc, h=h, w=w),
        grid=(batch, nchunk),
        in_specs=[spec(0), spec(1), spec(2), spec(3),
                  spec(0), spec(1), spec(2), spec(3)],
        out_specs=pl.BlockSpec((1, 1, 1), lambda b, j: (b, 0, 0),
                               memory_space=pltpu.SMEM),
        out_shape=jax.ShapeDtypeStruct((batch, 1, 1), jnp.float32),
        scratch_shapes=[
            pltpu.VMEM((8, w), jnp.float32),
            pltpu.VMEM((8, w), jnp.float32),
            pltpu.VMEM((8, w), jnp.float32),
            pltpu.VMEM((8, w), jnp.float32),
        ],
        compiler_params=pltpu.CompilerParams(
            dimension_semantics=("parallel", "arbitrary")),
    )(x, x, x, x, t, t, t, t)
    return jnp.mean(out)
